# Initial kernel scaffold; baseline (speedup 1.0000x reference)
import jax
import jax.numpy as jnp
from jax.experimental import pallas as pl


def kernel(x, edge_index, batch, hidden_channels, num_layers, W1, b1, g1, be1, W2, b2, g2, be2, W3, b3, g3, be3, linW, linb):
    raise NotImplementedError("write your pallas kernel here")


# speedup vs baseline: 18.4937x; 18.4937x over previous
import functools

import jax
import jax.numpy as jnp
from jax import lax
from jax.experimental import pallas as pl
from jax.experimental.pallas import tpu as pltpu
from jax.experimental.pallas import tpu_sc as plsc

N = 10000
E = 320000
D = 128
G = 64
C = 10

NC = 2
NS = 16

DEG_E = 327680
DEG_PAD = DEG_E - E
DEG_ACC = 10240
DEG_TILE_E = DEG_E // (NC * NS)

EP = 327680
EDGES_PER_TILE = EP // (NC * NS)
CHUNK_E = 256
N_CHUNKS = EDGES_PER_TILE // CHUNK_E
NPAD = 10240
ZROWS = 128


def _deg_body(dstp_hbm, degp_hbm, idx_v, vals_v, zsrc_v, acc_sh):
  c = lax.axis_index("c")
  s = lax.axis_index("s")

  @pl.loop(0, (DEG_ACC // NS) // 16)
  def _(i):
    zsrc_v[pl.ds(i * 16, 16)] = jnp.zeros((16,), jnp.float32)

  pltpu.sync_copy(zsrc_v, acc_sh.at[pl.ds(s * (DEG_ACC // NS), DEG_ACC // NS)])
  plsc.subcore_barrier()

  @pl.loop(0, DEG_TILE_E // 16)
  def _(i):
    vals_v[pl.ds(i * 16, 16)] = jnp.full((16,), 1.0, jnp.float32)

  e0 = c * (DEG_E // NC) + s * DEG_TILE_E
  pltpu.sync_copy(dstp_hbm.at[pl.ds(e0, DEG_TILE_E)], idx_v)
  pltpu.sync_copy(vals_v, acc_sh.at[idx_v], add=True)
  plsc.subcore_barrier()

  @pl.when(s == 0)
  def _():
    pltpu.sync_copy(acc_sh, degp_hbm.at[pl.ds(c * DEG_ACC, DEG_ACC)])


def _agg_body(src_hbm, dst_hbm, hs_hbm, outp_hbm, sidx_v, didx_v, rows_v,
              acc_sh):
  c = lax.axis_index("c")
  s = lax.axis_index("s")

  @pl.loop(0, ZROWS * 8)
  def _(i):
    rows_v[i // 8, pl.ds((i % 8) * 16, 16)] = jnp.zeros((16,), jnp.float32)

  @pl.loop(0, 640 // ZROWS)
  def _(i):
    pltpu.sync_copy(rows_v.at[pl.ds(0, ZROWS)],
                    acc_sh.at[pl.ds(s * 640 + i * ZROWS, ZROWS)])

  plsc.subcore_barrier()

  base_e = (c * NS + s) * EDGES_PER_TILE

  @pl.loop(0, N_CHUNKS)
  def _(k):
    e0 = base_e + k * CHUNK_E
    pltpu.sync_copy(src_hbm.at[pl.ds(e0, CHUNK_E)], sidx_v)
    pltpu.sync_copy(dst_hbm.at[pl.ds(e0, CHUNK_E)], didx_v)
    pltpu.sync_copy(hs_hbm.at[sidx_v], rows_v)
    pltpu.sync_copy(rows_v, acc_sh.at[didx_v], add=True)

  plsc.subcore_barrier()
  pltpu.sync_copy(acc_sh.at[pl.ds(s * 640, 640)],
                  outp_hbm.at[c, pl.ds(s * 640, 640)])


def _sc_mesh():
  return plsc.VectorSubcoreMesh(
      core_axis_name="c", subcore_axis_name="s", num_cores=NC, num_subcores=NS
  )


def _sc_degrees(dstp):
  k = pl.kernel(
      _deg_body,
      out_type=jax.ShapeDtypeStruct((NC * DEG_ACC,), jnp.float32),
      mesh=_sc_mesh(),
      scratch_types=[
          pltpu.VMEM((DEG_TILE_E,), jnp.int32),
          pltpu.VMEM((DEG_TILE_E,), jnp.float32),
          pltpu.VMEM((DEG_ACC // NS,), jnp.float32),
          pltpu.VMEM_SHARED((DEG_ACC,), jnp.float32),
      ],
  )
  return k(dstp)


def _sc_aggregate(src, dst, hs):
  k = pl.kernel(
      _agg_body,
      out_type=jax.ShapeDtypeStruct((NC, NPAD, D), jnp.float32),
      mesh=_sc_mesh(),
      scratch_types=[
          pltpu.VMEM((CHUNK_E,), jnp.int32),
          pltpu.VMEM((CHUNK_E,), jnp.int32),
          pltpu.VMEM((CHUNK_E, D), jnp.float32),
          pltpu.VMEM_SHARED((NPAD, D), jnp.float32),
      ],
  )
  return k(src, dst, hs)


def _dot(a, b):
  return lax.dot_general(
      a, b, (((1,), (0,)), ((), ())),
      precision=lax.Precision.HIGHEST,
      preferred_element_type=jnp.float32,
  )


def _mm_body(x_ref, w_ref, h_ref):
  h_ref[...] = _dot(x_ref[...], w_ref[...])


def _scale_body(degt_ref, h_ref, dinv_ref, hs_ref):
  deg = degt_ref[:, 0:1] + degt_ref[:, 1:2] + 1.0
  dinv = lax.rsqrt(deg)
  dinv_ref[...] = dinv
  hs_ref[...] = h_ref[...] * dinv


def _mid_body(aggp_ref, hs_ref, dinv_ref, b_ref, g_ref, be_ref, w_ref,
              hs2_ref, *, relu):
  dinv = dinv_ref[...]
  agg = aggp_ref[0, :N] + aggp_ref[1, :N] + hs_ref[...]
  conv = dinv * agg + b_ref[...]
  a = jnp.maximum(conv, 0.0) if relu else conv
  m = jnp.mean(a, axis=0, keepdims=True)
  v = jnp.mean((a - m) ** 2, axis=0, keepdims=True)
  z = (a - m) * lax.rsqrt(v + 1e-5) * g_ref[...] + be_ref[...]
  hs2_ref[...] = _dot(z, w_ref[...]) * dinv


def _final_body(aggp_ref, hs_ref, dinv_ref, b_ref, g_ref, be_ref,
                batch_ref, linw_ref, linb_ref, out_ref):
  dinv = dinv_ref[...]
  agg = aggp_ref[0, :N] + aggp_ref[1, :N] + hs_ref[...]
  conv = dinv * agg + b_ref[...]
  m = jnp.mean(conv, axis=0, keepdims=True)
  v = jnp.mean((conv - m) ** 2, axis=0, keepdims=True)
  z = (conv - m) * lax.rsqrt(v + 1e-5) * g_ref[...] + be_ref[...]
  seg = lax.broadcasted_iota(jnp.int32, (G, N), 0)
  p = (seg == jnp.broadcast_to(batch_ref[...], (G, N))).astype(jnp.float32)
  sums = _dot(p, z)
  counts = jnp.sum(p, axis=1, keepdims=True)
  pooled = sums / jnp.maximum(counts, 1.0)
  out_ref[...] = _dot(pooled, linw_ref[...]) + linb_ref[...]


def _tc_call(body, out_shapes):
  return pl.pallas_call(body, out_shape=out_shapes)


def kernel(x, edge_index, batch, hidden_channels, num_layers,
           W1, b1, g1, be1, W2, b2, g2, be2, W3, b3, g3, be3, linW, linb):
  del hidden_channels, num_layers
  src = edge_index[0]
  dst = edge_index[1]

  padi = jnp.arange(DEG_PAD, dtype=jnp.int32) % (DEG_ACC - N)
  dstp = jnp.concatenate([dst, N + padi])
  srcp = jnp.concatenate([src, padi])

  degp = _sc_degrees(dstp).reshape(NC, DEG_ACC)
  degt = degp[:, :N].T

  h1 = _tc_call(_mm_body, jax.ShapeDtypeStruct((N, D), jnp.float32))(x, W1)
  dinv, hs1 = _tc_call(
      _scale_body,
      (jax.ShapeDtypeStruct((N, 1), jnp.float32),
       jax.ShapeDtypeStruct((N, D), jnp.float32)),
  )(degt, h1)

  b1r, g1r, be1r = b1.reshape(1, D), g1.reshape(1, D), be1.reshape(1, D)
  b2r, g2r, be2r = b2.reshape(1, D), g2.reshape(1, D), be2.reshape(1, D)
  b3r, g3r, be3r = b3.reshape(1, D), g3.reshape(1, D), be3.reshape(1, D)

  agg1p = _sc_aggregate(srcp, dstp, hs1)
  hs2 = _tc_call(
      functools.partial(_mid_body, relu=True),
      jax.ShapeDtypeStruct((N, D), jnp.float32),
  )(agg1p, hs1, dinv, b1r, g1r, be1r, W2)

  agg2p = _sc_aggregate(srcp, dstp, hs2)
  hs3 = _tc_call(
      functools.partial(_mid_body, relu=True),
      jax.ShapeDtypeStruct((N, D), jnp.float32),
  )(agg2p, hs2, dinv, b2r, g2r, be2r, W3)

  agg3p = _sc_aggregate(srcp, dstp, hs3)
  out = _tc_call(
      _final_body, jax.ShapeDtypeStruct((G, C), jnp.float32),
  )(agg3p, hs3, dinv, b3r, g3r, be3r,
    batch.reshape(1, N), linW, linb.reshape(1, C))
  return out


# speedup vs baseline: 24.8581x; 1.3441x over previous
import functools

import jax
import jax.numpy as jnp
from jax import lax
from jax.experimental import pallas as pl
from jax.experimental.pallas import tpu as pltpu
from jax.experimental.pallas import tpu_sc as plsc

N = 10000
E = 320000
D = 128
G = 64
C = 10

NC = 2
NS = 16

DEG_E = 327680
DEG_PAD = DEG_E - E
DEG_ACC = 10240
DEG_TILE_E = DEG_E // (NC * NS)

EP = 327680
EDGES_PER_TILE = EP // (NC * NS)
CHUNK_E = 128
N_PAIRS = EDGES_PER_TILE // (2 * CHUNK_E)
NPAD = 10240
ZROWS = 128


def _deg_body(dstp_hbm, degp_hbm, idx_v, vals_v, zsrc_v, acc_sh):
  c = lax.axis_index("c")
  s = lax.axis_index("s")

  @pl.loop(0, (DEG_ACC // NS) // 16)
  def _(i):
    zsrc_v[pl.ds(i * 16, 16)] = jnp.zeros((16,), jnp.float32)

  pltpu.sync_copy(zsrc_v, acc_sh.at[pl.ds(s * (DEG_ACC // NS), DEG_ACC // NS)])
  plsc.subcore_barrier()

  @pl.loop(0, DEG_TILE_E // 16)
  def _(i):
    vals_v[pl.ds(i * 16, 16)] = jnp.full((16,), 1.0, jnp.float32)

  e0 = c * (DEG_E // NC) + s * DEG_TILE_E
  pltpu.sync_copy(dstp_hbm.at[pl.ds(e0, DEG_TILE_E)], idx_v)
  pltpu.sync_copy(vals_v, acc_sh.at[idx_v], add=True)
  plsc.subcore_barrier()

  @pl.when(s == 0)
  def _():
    pltpu.sync_copy(acc_sh, degp_hbm.at[pl.ds(c * DEG_ACC, DEG_ACC)])


def _agg_body(src_hbm, dst_hbm, hs_hbm, outp_hbm, sidx_v, didx_v, rows_v,
              isems, gsems, ssems, acc_sh):
  c = lax.axis_index("c")
  s = lax.axis_index("s")

  @pl.loop(0, ZROWS * 8)
  def _(i):
    rows_v[0, i // 8, pl.ds((i % 8) * 16, 16)] = jnp.zeros((16,), jnp.float32)

  @pl.loop(0, 640 // ZROWS)
  def _(i):
    pltpu.sync_copy(rows_v.at[0],
                    acc_sh.at[pl.ds(s * 640 + i * ZROWS, ZROWS)])

  plsc.subcore_barrier()

  base_e = (c * NS + s) * EDGES_PER_TILE

  def issue_idx(e0, b):
    pltpu.async_copy(src_hbm.at[pl.ds(e0, CHUNK_E)], sidx_v.at[b], isems.at[b])
    pltpu.async_copy(dst_hbm.at[pl.ds(e0, CHUNK_E)], didx_v.at[b], isems.at[b])

  def wait_idx(b):
    pltpu.make_async_copy(src_hbm.at[pl.ds(0, CHUNK_E)], sidx_v.at[b],
                          isems.at[b]).wait()
    pltpu.make_async_copy(dst_hbm.at[pl.ds(0, CHUNK_E)], didx_v.at[b],
                          isems.at[b]).wait()

  def gather(b):
    pltpu.async_copy(hs_hbm.at[sidx_v.at[b]], rows_v.at[b], gsems.at[b])

  def wait_gather(b):
    pltpu.make_async_copy(hs_hbm.at[sidx_v.at[b]], rows_v.at[b],
                          gsems.at[b]).wait()

  def scatter(b):
    pltpu.async_copy(rows_v.at[b], acc_sh.at[didx_v.at[b]], ssems.at[b],
                     add=True)

  def wait_scatter(b):
    pltpu.make_async_copy(rows_v.at[b], acc_sh.at[didx_v.at[b]],
                          ssems.at[b]).wait()

  issue_idx(base_e, 0)
  issue_idx(base_e + CHUNK_E, 1)
  wait_idx(0)
  gather(0)

  @pl.loop(0, N_PAIRS)
  def _(p):
    e0 = base_e + p * (2 * CHUNK_E)
    not_last = p < N_PAIRS - 1

    wait_gather(0)
    scatter(0)
    wait_idx(1)
    gather(1)
    wait_scatter(0)

    @pl.when(not_last)
    def _():
      issue_idx(e0 + 2 * CHUNK_E, 0)

    wait_gather(1)
    scatter(1)

    @pl.when(not_last)
    def _():
      wait_idx(0)
      gather(0)

    wait_scatter(1)

    @pl.when(not_last)
    def _():
      issue_idx(e0 + 3 * CHUNK_E, 1)

  plsc.subcore_barrier()
  pltpu.sync_copy(acc_sh.at[pl.ds(s * 640, 640)],
                  outp_hbm.at[c, pl.ds(s * 640, 640)])


def _sc_mesh():
  return plsc.VectorSubcoreMesh(
      core_axis_name="c", subcore_axis_name="s", num_cores=NC, num_subcores=NS
  )


def _sc_degrees(dstp):
  k = pl.kernel(
      _deg_body,
      out_type=jax.ShapeDtypeStruct((NC * DEG_ACC,), jnp.float32),
      mesh=_sc_mesh(),
      scratch_types=[
          pltpu.VMEM((DEG_TILE_E,), jnp.int32),
          pltpu.VMEM((DEG_TILE_E,), jnp.float32),
          pltpu.VMEM((DEG_ACC // NS,), jnp.float32),
          pltpu.VMEM_SHARED((DEG_ACC,), jnp.float32),
      ],
  )
  return k(dstp)


def _sc_aggregate(src, dst, hs):
  k = pl.kernel(
      _agg_body,
      out_type=jax.ShapeDtypeStruct((NC, NPAD, D), jnp.float32),
      mesh=_sc_mesh(),
      scratch_types=[
          pltpu.VMEM((2, CHUNK_E), jnp.int32),
          pltpu.VMEM((2, CHUNK_E), jnp.int32),
          pltpu.VMEM((2, CHUNK_E, D), jnp.float32),
          pltpu.SemaphoreType.DMA((2,)),
          pltpu.SemaphoreType.DMA((2,)),
          pltpu.SemaphoreType.DMA((2,)),
          pltpu.VMEM_SHARED((NPAD, D), jnp.float32),
      ],
  )
  return k(src, dst, hs)


def _dot(a, b):
  return lax.dot_general(
      a, b, (((1,), (0,)), ((), ())),
      precision=lax.Precision.HIGHEST,
      preferred_element_type=jnp.float32,
  )


def _mm_body(x_ref, w_ref, h_ref):
  h_ref[...] = _dot(x_ref[...], w_ref[...])


def _scale_body(degt_ref, h_ref, dinv_ref, hs_ref):
  deg = degt_ref[:, 0:1] + degt_ref[:, 1:2] + 1.0
  dinv = lax.rsqrt(deg)
  dinv_ref[...] = dinv
  hs_ref[...] = h_ref[...] * dinv


def _mid_body(aggp_ref, hs_ref, dinv_ref, b_ref, g_ref, be_ref, w_ref,
              hs2_ref, *, relu):
  dinv = dinv_ref[...]
  agg = aggp_ref[0, :N] + aggp_ref[1, :N] + hs_ref[...]
  conv = dinv * agg + b_ref[...]
  a = jnp.maximum(conv, 0.0) if relu else conv
  m = jnp.mean(a, axis=0, keepdims=True)
  v = jnp.mean((a - m) ** 2, axis=0, keepdims=True)
  z = (a - m) * lax.rsqrt(v + 1e-5) * g_ref[...] + be_ref[...]
  hs2_ref[...] = _dot(z, w_ref[...]) * dinv


def _final_body(aggp_ref, hs_ref, dinv_ref, b_ref, g_ref, be_ref,
                batch_ref, linw_ref, linb_ref, out_ref):
  dinv = dinv_ref[...]
  agg = aggp_ref[0, :N] + aggp_ref[1, :N] + hs_ref[...]
  conv = dinv * agg + b_ref[...]
  m = jnp.mean(conv, axis=0, keepdims=True)
  v = jnp.mean((conv - m) ** 2, axis=0, keepdims=True)
  z = (conv - m) * lax.rsqrt(v + 1e-5) * g_ref[...] + be_ref[...]
  seg = lax.broadcasted_iota(jnp.int32, (G, N), 0)
  p = (seg == jnp.broadcast_to(batch_ref[...], (G, N))).astype(jnp.float32)
  sums = _dot(p, z)
  counts = jnp.sum(p, axis=1, keepdims=True)
  pooled = sums / jnp.maximum(counts, 1.0)
  out_ref[...] = _dot(pooled, linw_ref[...]) + linb_ref[...]


def _tc_call(body, out_shapes):
  return pl.pallas_call(body, out_shape=out_shapes)


def kernel(x, edge_index, batch, hidden_channels, num_layers,
           W1, b1, g1, be1, W2, b2, g2, be2, W3, b3, g3, be3, linW, linb):
  del hidden_channels, num_layers
  src = edge_index[0]
  dst = edge_index[1]

  padi = jnp.arange(DEG_PAD, dtype=jnp.int32) % (DEG_ACC - N)
  dstp = jnp.concatenate([dst, N + padi])
  srcp = jnp.concatenate([src, padi])

  degp = _sc_degrees(dstp).reshape(NC, DEG_ACC)
  degt = degp[:, :N].T

  h1 = _tc_call(_mm_body, jax.ShapeDtypeStruct((N, D), jnp.float32))(x, W1)
  dinv, hs1 = _tc_call(
      _scale_body,
      (jax.ShapeDtypeStruct((N, 1), jnp.float32),
       jax.ShapeDtypeStruct((N, D), jnp.float32)),
  )(degt, h1)

  b1r, g1r, be1r = b1.reshape(1, D), g1.reshape(1, D), be1.reshape(1, D)
  b2r, g2r, be2r = b2.reshape(1, D), g2.reshape(1, D), be2.reshape(1, D)
  b3r, g3r, be3r = b3.reshape(1, D), g3.reshape(1, D), be3.reshape(1, D)

  agg1p = _sc_aggregate(srcp, dstp, hs1)
  hs2 = _tc_call(
      functools.partial(_mid_body, relu=True),
      jax.ShapeDtypeStruct((N, D), jnp.float32),
  )(agg1p, hs1, dinv, b1r, g1r, be1r, W2)

  agg2p = _sc_aggregate(srcp, dstp, hs2)
  hs3 = _tc_call(
      functools.partial(_mid_body, relu=True),
      jax.ShapeDtypeStruct((N, D), jnp.float32),
  )(agg2p, hs2, dinv, b2r, g2r, be2r, W3)

  agg3p = _sc_aggregate(srcp, dstp, hs3)
  out = _tc_call(
      _final_body, jax.ShapeDtypeStruct((G, C), jnp.float32),
  )(agg3p, hs3, dinv, b3r, g3r, be3r,
    batch.reshape(1, N), linW, linb.reshape(1, C))
  return out


# speedup vs baseline: 24.8906x; 1.0013x over previous
import functools

import jax
import jax.numpy as jnp
from jax import lax
from jax.experimental import pallas as pl
from jax.experimental.pallas import tpu as pltpu
from jax.experimental.pallas import tpu_sc as plsc

N = 10000
E = 320000
D = 128
G = 64
C = 10

NC = 2
NS = 16

DEG_E = 327680
DEG_PAD = DEG_E - E
DEG_ACC = 10240
DEG_TILE_E = DEG_E // (NC * NS)

EP = 327680
EDGES_PER_TILE = EP // (NC * NS)
CHUNK_E = 128
N_CHUNKS = EDGES_PER_TILE // CHUNK_E
N_QUADS = N_CHUNKS // 4
NPAD = 10240
ZROWS = 128


def _deg_body(dstp_hbm, degp_hbm, idx_v, vals_v, zsrc_v, acc_sh):
  c = lax.axis_index("c")
  s = lax.axis_index("s")

  @pl.loop(0, (DEG_ACC // NS) // 16)
  def _(i):
    zsrc_v[pl.ds(i * 16, 16)] = jnp.zeros((16,), jnp.float32)

  pltpu.sync_copy(zsrc_v, acc_sh.at[pl.ds(s * (DEG_ACC // NS), DEG_ACC // NS)])
  plsc.subcore_barrier()

  @pl.loop(0, DEG_TILE_E // 16)
  def _(i):
    vals_v[pl.ds(i * 16, 16)] = jnp.full((16,), 1.0, jnp.float32)

  e0 = c * (DEG_E // NC) + s * DEG_TILE_E
  pltpu.sync_copy(dstp_hbm.at[pl.ds(e0, DEG_TILE_E)], idx_v)
  pltpu.sync_copy(vals_v, acc_sh.at[idx_v], add=True)
  plsc.subcore_barrier()

  @pl.when(s == 0)
  def _():
    pltpu.sync_copy(acc_sh, degp_hbm.at[pl.ds(c * DEG_ACC, DEG_ACC)])


def _agg_body(src_hbm, dst_hbm, hs_hbm, outp_hbm, sidx_v, didx_v, rows_v,
              isems, gsems, ssems, acc_sh):
  c = lax.axis_index("c")
  s = lax.axis_index("s")

  @pl.loop(0, ZROWS * 8)
  def _(i):
    rows_v[0, i // 8, pl.ds((i % 8) * 16, 16)] = jnp.zeros((16,), jnp.float32)

  @pl.loop(0, 640 // ZROWS)
  def _(i):
    pltpu.sync_copy(rows_v.at[0, pl.ds(0, ZROWS)],
                    acc_sh.at[pl.ds(s * 640 + i * ZROWS, ZROWS)])

  plsc.subcore_barrier()

  base_e = (c * NS + s) * EDGES_PER_TILE

  def issue_idx(e0, b):
    pltpu.async_copy(src_hbm.at[pl.ds(e0, CHUNK_E)], sidx_v.at[b], isems.at[b])
    pltpu.async_copy(dst_hbm.at[pl.ds(e0, CHUNK_E)], didx_v.at[b], isems.at[b])

  def wait_idx(b):
    pltpu.make_async_copy(src_hbm.at[pl.ds(0, CHUNK_E)], sidx_v.at[b],
                          isems.at[b]).wait()
    pltpu.make_async_copy(dst_hbm.at[pl.ds(0, CHUNK_E)], didx_v.at[b],
                          isems.at[b]).wait()

  def gather(r, i):
    pltpu.async_copy(hs_hbm.at[sidx_v.at[i]], rows_v.at[r], gsems.at[r])

  def wait_gather(r, i):
    pltpu.make_async_copy(hs_hbm.at[sidx_v.at[i]], rows_v.at[r],
                          gsems.at[r]).wait()

  def scatter(r, i):
    pltpu.async_copy(rows_v.at[r], acc_sh.at[didx_v.at[i]], ssems.at[r],
                     add=True)

  def wait_scatter(r, i):
    pltpu.make_async_copy(rows_v.at[r], acc_sh.at[didx_v.at[i]],
                          ssems.at[r]).wait()

  issue_idx(base_e, 0)
  issue_idx(base_e + CHUNK_E, 1)
  issue_idx(base_e + 2 * CHUNK_E, 2)
  wait_idx(0)
  gather(0, 0)

  @pl.loop(0, N_QUADS)
  def _(q):
    e0 = base_e + q * (4 * CHUNK_E)
    not_last = q < N_QUADS - 1

    def step(t, guard_first=False, do_prefetch=True, do_next_gather=True):
      j_r, j_i = t % 2, t % 4
      p_r, p_i = (t + 1) % 2, (t + 3) % 4

      if guard_first:
        @pl.when(q > 0)
        def _():
          wait_scatter(p_r, p_i)
      else:
        wait_scatter(p_r, p_i)

      if do_prefetch is True:
        issue_idx(e0 + (t + 3) * CHUNK_E, (t + 3) % 4)
      elif do_prefetch is not False:
        @pl.when(do_prefetch)
        def _():
          issue_idx(e0 + (t + 3) * CHUNK_E, (t + 3) % 4)

      wait_gather(j_r, j_i)
      scatter(j_r, j_i)

      if do_next_gather is True:
        wait_idx((t + 1) % 4)
        gather((t + 1) % 2, (t + 1) % 4)
      elif do_next_gather is not False:
        @pl.when(do_next_gather)
        def _():
          wait_idx((t + 1) % 4)
          gather((t + 1) % 2, (t + 1) % 4)

    step(0, guard_first=True)
    step(1, do_prefetch=not_last)
    step(2, do_prefetch=not_last)
    step(3, do_prefetch=not_last, do_next_gather=not_last)

  wait_scatter(1, 3)
  plsc.subcore_barrier()
  pltpu.sync_copy(acc_sh.at[pl.ds(s * 640, 640)],
                  outp_hbm.at[c, pl.ds(s * 640, 640)])


def _sc_mesh():
  return plsc.VectorSubcoreMesh(
      core_axis_name="c", subcore_axis_name="s", num_cores=NC, num_subcores=NS
  )


def _sc_degrees(dstp):
  k = pl.kernel(
      _deg_body,
      out_type=jax.ShapeDtypeStruct((NC * DEG_ACC,), jnp.float32),
      mesh=_sc_mesh(),
      scratch_types=[
          pltpu.VMEM((DEG_TILE_E,), jnp.int32),
          pltpu.VMEM((DEG_TILE_E,), jnp.float32),
          pltpu.VMEM((DEG_ACC // NS,), jnp.float32),
          pltpu.VMEM_SHARED((DEG_ACC,), jnp.float32),
      ],
  )
  return k(dstp)


def _sc_aggregate(src, dst, hs):
  k = pl.kernel(
      _agg_body,
      out_type=jax.ShapeDtypeStruct((NC, NPAD, D), jnp.float32),
      mesh=_sc_mesh(),
      scratch_types=[
          pltpu.VMEM((4, CHUNK_E), jnp.int32),
          pltpu.VMEM((4, CHUNK_E), jnp.int32),
          pltpu.VMEM((2, CHUNK_E, D), jnp.float32),
          pltpu.SemaphoreType.DMA((4,)),
          pltpu.SemaphoreType.DMA((2,)),
          pltpu.SemaphoreType.DMA((2,)),
          pltpu.VMEM_SHARED((NPAD, D), jnp.float32),
      ],
  )
  return k(src, dst, hs)


def _dot(a, b):
  return lax.dot_general(
      a, b, (((1,), (0,)), ((), ())),
      precision=lax.Precision.HIGHEST,
      preferred_element_type=jnp.float32,
  )


def _mm_body(x_ref, w_ref, h_ref):
  h_ref[...] = _dot(x_ref[...], w_ref[...])


def _scale_body(degt_ref, h_ref, dinv_ref, hs_ref):
  deg = degt_ref[:, 0:1] + degt_ref[:, 1:2] + 1.0
  dinv = lax.rsqrt(deg)
  dinv_ref[...] = dinv
  hs_ref[...] = h_ref[...] * dinv


def _mid_body(aggp_ref, hs_ref, dinv_ref, b_ref, g_ref, be_ref, w_ref,
              hs2_ref, *, relu):
  dinv = dinv_ref[...]
  agg = aggp_ref[0, :N] + aggp_ref[1, :N] + hs_ref[...]
  conv = dinv * agg + b_ref[...]
  a = jnp.maximum(conv, 0.0) if relu else conv
  m = jnp.mean(a, axis=0, keepdims=True)
  v = jnp.mean((a - m) ** 2, axis=0, keepdims=True)
  z = (a - m) * lax.rsqrt(v + 1e-5) * g_ref[...] + be_ref[...]
  hs2_ref[...] = _dot(z, w_ref[...]) * dinv


def _final_body(aggp_ref, hs_ref, dinv_ref, b_ref, g_ref, be_ref,
                batch_ref, linw_ref, linb_ref, out_ref):
  dinv = dinv_ref[...]
  agg = aggp_ref[0, :N] + aggp_ref[1, :N] + hs_ref[...]
  conv = dinv * agg + b_ref[...]
  m = jnp.mean(conv, axis=0, keepdims=True)
  v = jnp.mean((conv - m) ** 2, axis=0, keepdims=True)
  z = (conv - m) * lax.rsqrt(v + 1e-5) * g_ref[...] + be_ref[...]
  seg = lax.broadcasted_iota(jnp.int32, (G, N), 0)
  p = (seg == jnp.broadcast_to(batch_ref[...], (G, N))).astype(jnp.float32)
  sums = _dot(p, z)
  counts = jnp.sum(p, axis=1, keepdims=True)
  pooled = sums / jnp.maximum(counts, 1.0)
  out_ref[...] = _dot(pooled, linw_ref[...]) + linb_ref[...]


def _tc_call(body, out_shapes):
  return pl.pallas_call(body, out_shape=out_shapes)


def kernel(x, edge_index, batch, hidden_channels, num_layers,
           W1, b1, g1, be1, W2, b2, g2, be2, W3, b3, g3, be3, linW, linb):
  del hidden_channels, num_layers
  src = edge_index[0]
  dst = edge_index[1]

  padi = jnp.arange(DEG_PAD, dtype=jnp.int32) % (DEG_ACC - N)
  dstp = jnp.concatenate([dst, N + padi])
  srcp = jnp.concatenate([src, padi])

  degp = _sc_degrees(dstp).reshape(NC, DEG_ACC)
  degt = degp[:, :N].T

  h1 = _tc_call(_mm_body, jax.ShapeDtypeStruct((N, D), jnp.float32))(x, W1)
  dinv, hs1 = _tc_call(
      _scale_body,
      (jax.ShapeDtypeStruct((N, 1), jnp.float32),
       jax.ShapeDtypeStruct((N, D), jnp.float32)),
  )(degt, h1)

  b1r, g1r, be1r = b1.reshape(1, D), g1.reshape(1, D), be1.reshape(1, D)
  b2r, g2r, be2r = b2.reshape(1, D), g2.reshape(1, D), be2.reshape(1, D)
  b3r, g3r, be3r = b3.reshape(1, D), g3.reshape(1, D), be3.reshape(1, D)

  agg1p = _sc_aggregate(srcp, dstp, hs1)
  hs2 = _tc_call(
      functools.partial(_mid_body, relu=True),
      jax.ShapeDtypeStruct((N, D), jnp.float32),
  )(agg1p, hs1, dinv, b1r, g1r, be1r, W2)

  agg2p = _sc_aggregate(srcp, dstp, hs2)
  hs3 = _tc_call(
      functools.partial(_mid_body, relu=True),
      jax.ShapeDtypeStruct((N, D), jnp.float32),
  )(agg2p, hs2, dinv, b2r, g2r, be2r, W3)

  agg3p = _sc_aggregate(srcp, dstp, hs3)
  out = _tc_call(
      _final_body, jax.ShapeDtypeStruct((G, C), jnp.float32),
  )(agg3p, hs3, dinv, b3r, g3r, be3r,
    batch.reshape(1, N), linW, linb.reshape(1, C))
  return out


# speedup vs baseline: 24.8990x; 1.0003x over previous
import functools

import jax
import jax.numpy as jnp
from jax import lax
from jax.experimental import pallas as pl
from jax.experimental.pallas import tpu as pltpu
from jax.experimental.pallas import tpu_sc as plsc

N = 10000
E = 320000
D = 128
G = 64
C = 10

NC = 2
NS = 16

DEG_E = 327680
DEG_PAD = DEG_E - E
DEG_ACC = 10240
DEG_TILE_E = DEG_E // (NC * NS)

EP = 327680
EDGES_PER_TILE = EP // (NC * NS)
CHUNK_E = 128
N_CHUNKS = EDGES_PER_TILE // CHUNK_E
N_QUADS = N_CHUNKS // 4
NPAD = 10240
ZROWS = 128


def _deg_body(dstp_hbm, degp_hbm, idx_v, vals_v, zsrc_v, acc_sh):
  c = lax.axis_index("c")
  s = lax.axis_index("s")

  @pl.loop(0, (DEG_ACC // NS) // 16)
  def _(i):
    zsrc_v[pl.ds(i * 16, 16)] = jnp.zeros((16,), jnp.float32)

  pltpu.sync_copy(zsrc_v, acc_sh.at[pl.ds(s * (DEG_ACC // NS), DEG_ACC // NS)])
  plsc.subcore_barrier()

  @pl.loop(0, DEG_TILE_E // 16)
  def _(i):
    vals_v[pl.ds(i * 16, 16)] = jnp.full((16,), 1.0, jnp.float32)

  e0 = c * (DEG_E // NC) + s * DEG_TILE_E
  pltpu.sync_copy(dstp_hbm.at[pl.ds(e0, DEG_TILE_E)], idx_v)
  pltpu.sync_copy(vals_v, acc_sh.at[idx_v], add=True)
  plsc.subcore_barrier()

  @pl.when(s == 0)
  def _():
    pltpu.sync_copy(acc_sh, degp_hbm.at[pl.ds(c * DEG_ACC, DEG_ACC)])


def _agg_body(src_hbm, dst_hbm, hs_hbm, outp_hbm, sidx_v, didx_v, rows_v,
              isems, gsems, ssems, acc_sh):
  c = lax.axis_index("c")
  s = lax.axis_index("s")
  base_e = (c * NS + s) * EDGES_PER_TILE

  def issue_idx(e0, b):
    pltpu.async_copy(src_hbm.at[pl.ds(e0, CHUNK_E)], sidx_v.at[b], isems.at[b])
    pltpu.async_copy(dst_hbm.at[pl.ds(e0, CHUNK_E)], didx_v.at[b], isems.at[b])

  def wait_idx(b):
    pltpu.make_async_copy(src_hbm.at[pl.ds(0, CHUNK_E)], sidx_v.at[b],
                          isems.at[b]).wait()
    pltpu.make_async_copy(dst_hbm.at[pl.ds(0, CHUNK_E)], didx_v.at[b],
                          isems.at[b]).wait()

  def gather(r, i):
    pltpu.async_copy(hs_hbm.at[sidx_v.at[i]], rows_v.at[r], gsems.at[r])

  def wait_gather(r, i):
    pltpu.make_async_copy(hs_hbm.at[sidx_v.at[i]], rows_v.at[r],
                          gsems.at[r]).wait()

  def scatter(r, i):
    pltpu.async_copy(rows_v.at[r], acc_sh.at[didx_v.at[i]], ssems.at[r],
                     add=True)

  def wait_scatter(r, i):
    pltpu.make_async_copy(rows_v.at[r], acc_sh.at[didx_v.at[i]],
                          ssems.at[r]).wait()

  issue_idx(base_e, 0)
  issue_idx(base_e + CHUNK_E, 1)
  issue_idx(base_e + 2 * CHUNK_E, 2)
  wait_idx(0)
  gather(0, 0)

  @pl.loop(0, ZROWS * 8)
  def _(i):
    rows_v[1, i // 8, pl.ds((i % 8) * 16, 16)] = jnp.zeros((16,), jnp.float32)

  @pl.loop(0, 640 // ZROWS)
  def _(i):
    pltpu.sync_copy(rows_v.at[1, pl.ds(0, ZROWS)],
                    acc_sh.at[pl.ds(s * 640 + i * ZROWS, ZROWS)])

  plsc.subcore_barrier()

  @pl.loop(0, N_QUADS)
  def _(q):
    e0 = base_e + q * (4 * CHUNK_E)
    not_last = q < N_QUADS - 1

    def step(t, guard_first=False, do_prefetch=True, do_next_gather=True):
      j_r, j_i = t % 2, t % 4
      p_r, p_i = (t + 1) % 2, (t + 3) % 4

      if guard_first:
        @pl.when(q > 0)
        def _():
          wait_scatter(p_r, p_i)
      else:
        wait_scatter(p_r, p_i)

      if do_prefetch is True:
        issue_idx(e0 + (t + 3) * CHUNK_E, (t + 3) % 4)
      elif do_prefetch is not False:
        @pl.when(do_prefetch)
        def _():
          issue_idx(e0 + (t + 3) * CHUNK_E, (t + 3) % 4)

      wait_gather(j_r, j_i)
      scatter(j_r, j_i)

      if do_next_gather is True:
        wait_idx((t + 1) % 4)
        gather((t + 1) % 2, (t + 1) % 4)
      elif do_next_gather is not False:
        @pl.when(do_next_gather)
        def _():
          wait_idx((t + 1) % 4)
          gather((t + 1) % 2, (t + 1) % 4)

    step(0, guard_first=True)
    step(1, do_prefetch=not_last)
    step(2, do_prefetch=not_last)
    step(3, do_prefetch=not_last, do_next_gather=not_last)

  wait_scatter(1, 3)
  plsc.subcore_barrier()
  pltpu.sync_copy(acc_sh.at[pl.ds(s * 640, 640)],
                  outp_hbm.at[c, pl.ds(s * 640, 640)])


def _sc_mesh():
  return plsc.VectorSubcoreMesh(
      core_axis_name="c", subcore_axis_name="s", num_cores=NC, num_subcores=NS
  )


def _sc_degrees(dstp):
  k = pl.kernel(
      _deg_body,
      out_type=jax.ShapeDtypeStruct((NC * DEG_ACC,), jnp.float32),
      mesh=_sc_mesh(),
      scratch_types=[
          pltpu.VMEM((DEG_TILE_E,), jnp.int32),
          pltpu.VMEM((DEG_TILE_E,), jnp.float32),
          pltpu.VMEM((DEG_ACC // NS,), jnp.float32),
          pltpu.VMEM_SHARED((DEG_ACC,), jnp.float32),
      ],
  )
  return k(dstp)


def _sc_aggregate(src, dst, hs):
  k = pl.kernel(
      _agg_body,
      out_type=jax.ShapeDtypeStruct((NC, NPAD, D), jnp.float32),
      mesh=_sc_mesh(),
      scratch_types=[
          pltpu.VMEM((4, CHUNK_E), jnp.int32),
          pltpu.VMEM((4, CHUNK_E), jnp.int32),
          pltpu.VMEM((2, CHUNK_E, D), jnp.float32),
          pltpu.SemaphoreType.DMA((4,)),
          pltpu.SemaphoreType.DMA((2,)),
          pltpu.SemaphoreType.DMA((2,)),
          pltpu.VMEM_SHARED((NPAD, D), jnp.float32),
      ],
  )
  return k(src, dst, hs)


def _dot(a, b):
  return lax.dot_general(
      a, b, (((1,), (0,)), ((), ())),
      precision=lax.Precision.HIGHEST,
      preferred_element_type=jnp.float32,
  )


def _first_body(degt_ref, x_ref, w_ref, dinv_ref, hs_ref):
  deg = degt_ref[:, 0:1] + degt_ref[:, 1:2] + 1.0
  dinv = lax.rsqrt(deg)
  dinv_ref[...] = dinv
  hs_ref[...] = _dot(x_ref[...], w_ref[...]) * dinv


def _mid_body(aggp_ref, hs_ref, dinv_ref, b_ref, g_ref, be_ref, w_ref,
              hs2_ref, *, relu):
  dinv = dinv_ref[...]
  agg = aggp_ref[0, :N] + aggp_ref[1, :N] + hs_ref[...]
  conv = dinv * agg + b_ref[...]
  a = jnp.maximum(conv, 0.0) if relu else conv
  m = jnp.mean(a, axis=0, keepdims=True)
  v = jnp.mean((a - m) ** 2, axis=0, keepdims=True)
  z = (a - m) * lax.rsqrt(v + 1e-5) * g_ref[...] + be_ref[...]
  hs2_ref[...] = _dot(z, w_ref[...]) * dinv


def _final_body(aggp_ref, hs_ref, dinv_ref, b_ref, g_ref, be_ref,
                batch_ref, linw_ref, linb_ref, out_ref):
  dinv = dinv_ref[...]
  agg = aggp_ref[0, :N] + aggp_ref[1, :N] + hs_ref[...]
  conv = dinv * agg + b_ref[...]
  m = jnp.mean(conv, axis=0, keepdims=True)
  v = jnp.mean((conv - m) ** 2, axis=0, keepdims=True)
  z = (conv - m) * lax.rsqrt(v + 1e-5) * g_ref[...] + be_ref[...]
  seg = lax.broadcasted_iota(jnp.int32, (G, N), 0)
  p = (seg == jnp.broadcast_to(batch_ref[...], (G, N))).astype(jnp.float32)
  sums = _dot(p, z)
  counts = jnp.sum(p, axis=1, keepdims=True)
  pooled = sums / jnp.maximum(counts, 1.0)
  out_ref[...] = _dot(pooled, linw_ref[...]) + linb_ref[...]


def _tc_call(body, out_shapes):
  return pl.pallas_call(body, out_shape=out_shapes)


def kernel(x, edge_index, batch, hidden_channels, num_layers,
           W1, b1, g1, be1, W2, b2, g2, be2, W3, b3, g3, be3, linW, linb):
  del hidden_channels, num_layers
  src = edge_index[0]
  dst = edge_index[1]

  padi = jnp.arange(DEG_PAD, dtype=jnp.int32) % (DEG_ACC - N)
  dstp = jnp.concatenate([dst, N + padi])
  srcp = jnp.concatenate([src, padi])

  degp = _sc_degrees(dstp).reshape(NC, DEG_ACC)
  degt = degp[:, :N].T

  dinv, hs1 = _tc_call(
      _first_body,
      (jax.ShapeDtypeStruct((N, 1), jnp.float32),
       jax.ShapeDtypeStruct((N, D), jnp.float32)),
  )(degt, x, W1)

  b1r, g1r, be1r = b1.reshape(1, D), g1.reshape(1, D), be1.reshape(1, D)
  b2r, g2r, be2r = b2.reshape(1, D), g2.reshape(1, D), be2.reshape(1, D)
  b3r, g3r, be3r = b3.reshape(1, D), g3.reshape(1, D), be3.reshape(1, D)

  agg1p = _sc_aggregate(srcp, dstp, hs1)
  hs2 = _tc_call(
      functools.partial(_mid_body, relu=True),
      jax.ShapeDtypeStruct((N, D), jnp.float32),
  )(agg1p, hs1, dinv, b1r, g1r, be1r, W2)

  agg2p = _sc_aggregate(srcp, dstp, hs2)
  hs3 = _tc_call(
      functools.partial(_mid_body, relu=True),
      jax.ShapeDtypeStruct((N, D), jnp.float32),
  )(agg2p, hs2, dinv, b2r, g2r, be2r, W3)

  agg3p = _sc_aggregate(srcp, dstp, hs3)
  out = _tc_call(
      _final_body, jax.ShapeDtypeStruct((G, C), jnp.float32),
  )(agg3p, hs3, dinv, b3r, g3r, be3r,
    batch.reshape(1, N), linW, linb.reshape(1, C))
  return out


# speedup vs baseline: 24.9954x; 1.0039x over previous
import functools

import jax
import jax.numpy as jnp
from jax import lax
from jax.experimental import pallas as pl
from jax.experimental.pallas import tpu as pltpu
from jax.experimental.pallas import tpu_sc as plsc

N = 10000
E = 320000
D = 128
G = 64
C = 10

NC = 2
NS = 16

DEG_E = 327680
DEG_PAD = DEG_E - E
DEG_ACC = 10240
DEG_TILE_E = DEG_E // (NC * NS)

EP = 327680
EDGES_PER_TILE = EP // (NC * NS)
CHUNK_E = 128
N_CHUNKS = EDGES_PER_TILE // CHUNK_E
GROUP = 8
N_GROUPS = N_CHUNKS // GROUP
NPAD = 10240
ZROWS = 128


def _deg_body(dstp_hbm, degp_hbm, idx_v, vals_v, zsrc_v, acc_sh):
  c = lax.axis_index("c")
  s = lax.axis_index("s")

  @pl.loop(0, (DEG_ACC // NS) // 16)
  def _(i):
    zsrc_v[pl.ds(i * 16, 16)] = jnp.zeros((16,), jnp.float32)

  pltpu.sync_copy(zsrc_v, acc_sh.at[pl.ds(s * (DEG_ACC // NS), DEG_ACC // NS)])
  plsc.subcore_barrier()

  @pl.loop(0, DEG_TILE_E // 16)
  def _(i):
    vals_v[pl.ds(i * 16, 16)] = jnp.full((16,), 1.0, jnp.float32)

  e0 = c * (DEG_E // NC) + s * DEG_TILE_E
  pltpu.sync_copy(dstp_hbm.at[pl.ds(e0, DEG_TILE_E)], idx_v)
  pltpu.sync_copy(vals_v, acc_sh.at[idx_v], add=True)
  plsc.subcore_barrier()

  @pl.when(s == 0)
  def _():
    pltpu.sync_copy(acc_sh, degp_hbm.at[pl.ds(c * DEG_ACC, DEG_ACC)])


def _agg_body(src_hbm, dst_hbm, hs_hbm, outp_hbm, sidx_v, didx_v, rows_v,
              isems, gsems, ssems, acc_sh):
  c = lax.axis_index("c")
  s = lax.axis_index("s")
  base_e = (c * NS + s) * EDGES_PER_TILE

  base_r = base_e // CHUNK_E

  def issue_group(g_expr, b):
    r0 = pl.multiple_of(base_r + g_expr * GROUP, 8)
    pltpu.async_copy(src_hbm.at[pl.ds(r0, GROUP)], sidx_v.at[b], isems.at[b])
    pltpu.async_copy(dst_hbm.at[pl.ds(r0, GROUP)], didx_v.at[b], isems.at[b])

  def wait_group(b):
    pltpu.make_async_copy(src_hbm.at[pl.ds(0, GROUP)], sidx_v.at[b],
                          isems.at[b]).wait()
    pltpu.make_async_copy(dst_hbm.at[pl.ds(0, GROUP)], didx_v.at[b],
                          isems.at[b]).wait()

  def gather(r, b, jj):
    pltpu.async_copy(hs_hbm.at[sidx_v.at[b, jj]], rows_v.at[r], gsems.at[r])

  def wait_gather(r, b, jj):
    pltpu.make_async_copy(hs_hbm.at[sidx_v.at[b, jj]], rows_v.at[r],
                          gsems.at[r]).wait()

  def scatter(r, b, jj):
    pltpu.async_copy(rows_v.at[r], acc_sh.at[didx_v.at[b, jj]], ssems.at[r],
                     add=True)

  def wait_scatter(r, b, jj):
    pltpu.make_async_copy(rows_v.at[r], acc_sh.at[didx_v.at[b, jj]],
                          ssems.at[r]).wait()

  issue_group(0, 0)
  wait_group(0)
  gather(0, 0, 0)

  @pl.loop(0, ZROWS * 8)
  def _(i):
    rows_v[1, i // 8, pl.ds((i % 8) * 16, 16)] = jnp.zeros((16,), jnp.float32)

  @pl.loop(0, 640 // ZROWS)
  def _(i):
    pltpu.sync_copy(rows_v.at[1, pl.ds(0, ZROWS)],
                    acc_sh.at[pl.ds(s * 640 + i * ZROWS, ZROWS)])

  plsc.subcore_barrier()

  @pl.loop(0, N_GROUPS // 2)
  def _(u):
    not_last_u = u < N_GROUPS // 2 - 1

    def group_body(g_expr, bg, first_guard, prefetch_guard, next_guard):
      for jj in range(GROUP):
        r = jj % 2
        pr, pb, pjj = (jj + 1) % 2, (bg if jj > 0 else 1 - bg), (jj - 1) % GROUP
        if jj == 0 and first_guard is not None:
          @pl.when(first_guard)
          def _():
            wait_scatter(pr, pb, pjj)
        else:
          wait_scatter(pr, pb, pjj)

        if jj == 0:
          if prefetch_guard is None:
            issue_group(g_expr + 1, 1 - bg)
          else:
            @pl.when(prefetch_guard)
            def _():
              issue_group(g_expr + 1, 1 - bg)

        wait_gather(r, bg, jj)
        scatter(r, bg, jj)

        if jj < GROUP - 1:
          gather((jj + 1) % 2, bg, jj + 1)
        elif next_guard is None:
          wait_group(1 - bg)
          gather((jj + 1) % 2, 1 - bg, 0)
        else:
          @pl.when(next_guard)
          def _():
            wait_group(1 - bg)
            gather((jj + 1) % 2, 1 - bg, 0)

    g0 = 2 * u
    group_body(g0, 0, first_guard=(u > 0), prefetch_guard=None,
               next_guard=None)
    group_body(g0 + 1, 1, first_guard=None, prefetch_guard=not_last_u,
               next_guard=not_last_u)

  wait_scatter(1, 1, GROUP - 1)
  plsc.subcore_barrier()
  pltpu.sync_copy(acc_sh.at[pl.ds(s * 640, 640)],
                  outp_hbm.at[c, pl.ds(s * 640, 640)])


def _sc_mesh():
  return plsc.VectorSubcoreMesh(
      core_axis_name="c", subcore_axis_name="s", num_cores=NC, num_subcores=NS
  )


def _sc_degrees(dstp):
  k = pl.kernel(
      _deg_body,
      out_type=jax.ShapeDtypeStruct((NC * DEG_ACC,), jnp.float32),
      mesh=_sc_mesh(),
      scratch_types=[
          pltpu.VMEM((DEG_TILE_E,), jnp.int32),
          pltpu.VMEM((DEG_TILE_E,), jnp.float32),
          pltpu.VMEM((DEG_ACC // NS,), jnp.float32),
          pltpu.VMEM_SHARED((DEG_ACC,), jnp.float32),
      ],
  )
  return k(dstp)


def _sc_aggregate(src, dst, hs):
  k = pl.kernel(
      _agg_body,
      out_type=jax.ShapeDtypeStruct((NC, NPAD, D), jnp.float32),
      mesh=_sc_mesh(),
      scratch_types=[
          pltpu.VMEM((2, GROUP, CHUNK_E), jnp.int32),
          pltpu.VMEM((2, GROUP, CHUNK_E), jnp.int32),
          pltpu.VMEM((2, CHUNK_E, D), jnp.float32),
          pltpu.SemaphoreType.DMA((2,)),
          pltpu.SemaphoreType.DMA((2,)),
          pltpu.SemaphoreType.DMA((2,)),
          pltpu.VMEM_SHARED((NPAD, D), jnp.float32),
      ],
  )
  return k(src, dst, hs)


def _dot(a, b):
  return lax.dot_general(
      a, b, (((1,), (0,)), ((), ())),
      precision=lax.Precision.HIGHEST,
      preferred_element_type=jnp.float32,
  )


def _first_body(degt_ref, x_ref, w_ref, dinv_ref, hs_ref):
  deg = degt_ref[:, 0:1] + degt_ref[:, 1:2] + 1.0
  dinv = lax.rsqrt(deg)
  dinv_ref[...] = dinv
  hs_ref[...] = _dot(x_ref[...], w_ref[...]) * dinv


def _mid_body(aggp_ref, hs_ref, dinv_ref, b_ref, g_ref, be_ref, w_ref,
              hs2_ref, *, relu):
  dinv = dinv_ref[...]
  agg = aggp_ref[0, :N] + aggp_ref[1, :N] + hs_ref[...]
  conv = dinv * agg + b_ref[...]
  a = jnp.maximum(conv, 0.0) if relu else conv
  m = jnp.mean(a, axis=0, keepdims=True)
  v = jnp.mean((a - m) ** 2, axis=0, keepdims=True)
  z = (a - m) * lax.rsqrt(v + 1e-5) * g_ref[...] + be_ref[...]
  hs2_ref[...] = _dot(z, w_ref[...]) * dinv


def _final_body(aggp_ref, hs_ref, dinv_ref, b_ref, g_ref, be_ref,
                batch_ref, linw_ref, linb_ref, out_ref):
  dinv = dinv_ref[...]
  agg = aggp_ref[0, :N] + aggp_ref[1, :N] + hs_ref[...]
  conv = dinv * agg + b_ref[...]
  m = jnp.mean(conv, axis=0, keepdims=True)
  v = jnp.mean((conv - m) ** 2, axis=0, keepdims=True)
  z = (conv - m) * lax.rsqrt(v + 1e-5) * g_ref[...] + be_ref[...]
  seg = lax.broadcasted_iota(jnp.int32, (G, N), 0)
  p = (seg == jnp.broadcast_to(batch_ref[...], (G, N))).astype(jnp.float32)
  sums = _dot(p, z)
  counts = jnp.sum(p, axis=1, keepdims=True)
  pooled = sums / jnp.maximum(counts, 1.0)
  out_ref[...] = _dot(pooled, linw_ref[...]) + linb_ref[...]


def _tc_call(body, out_shapes):
  return pl.pallas_call(body, out_shape=out_shapes)


def kernel(x, edge_index, batch, hidden_channels, num_layers,
           W1, b1, g1, be1, W2, b2, g2, be2, W3, b3, g3, be3, linW, linb):
  del hidden_channels, num_layers
  src = edge_index[0]
  dst = edge_index[1]

  padi = jnp.arange(DEG_PAD, dtype=jnp.int32) % (DEG_ACC - N)
  dstp = jnp.concatenate([dst, N + padi])
  src2d = jnp.concatenate(
      [src.reshape(E // 128, 128), padi.reshape(DEG_PAD // 128, 128)])
  dst2d = jnp.concatenate(
      [dst.reshape(E // 128, 128), (N + padi).reshape(DEG_PAD // 128, 128)])

  degp = _sc_degrees(dstp).reshape(NC, DEG_ACC)
  degt = degp[:, :N].T

  dinv, hs1 = _tc_call(
      _first_body,
      (jax.ShapeDtypeStruct((N, 1), jnp.float32),
       jax.ShapeDtypeStruct((N, D), jnp.float32)),
  )(degt, x, W1)

  b1r, g1r, be1r = b1.reshape(1, D), g1.reshape(1, D), be1.reshape(1, D)
  b2r, g2r, be2r = b2.reshape(1, D), g2.reshape(1, D), be2.reshape(1, D)
  b3r, g3r, be3r = b3.reshape(1, D), g3.reshape(1, D), be3.reshape(1, D)

  agg1p = _sc_aggregate(src2d, dst2d, hs1)
  hs2 = _tc_call(
      functools.partial(_mid_body, relu=True),
      jax.ShapeDtypeStruct((N, D), jnp.float32),
  )(agg1p, hs1, dinv, b1r, g1r, be1r, W2)

  agg2p = _sc_aggregate(src2d, dst2d, hs2)
  hs3 = _tc_call(
      functools.partial(_mid_body, relu=True),
      jax.ShapeDtypeStruct((N, D), jnp.float32),
  )(agg2p, hs2, dinv, b2r, g2r, be2r, W3)

  agg3p = _sc_aggregate(src2d, dst2d, hs3)
  out = _tc_call(
      _final_body, jax.ShapeDtypeStruct((G, C), jnp.float32),
  )(agg3p, hs3, dinv, b3r, g3r, be3r,
    batch.reshape(1, N), linW, linb.reshape(1, C))
  return out
